# baseline (speedup 1.0000x reference)
import functools

import jax
import jax.numpy as jnp
from jax import lax
from jax.experimental import pallas as pl
from jax.experimental.pallas import tpu as pltpu
from jax.experimental.pallas import tpu_sc as plsc

_NC, _NS = 2, 16
_NW = _NC * _NS


def _make_sc_gather(B, F, Fd, V):
    CH = 32768
    V128 = (V // 128) * 128
    tail_b = V - V128
    H1 = ((V128 // 2 + 127) // 128) * 128
    H2 = V128 - H1
    B2 = H2 + tail_b
    mesh = plsc.VectorSubcoreMesh(
        core_axis_name="c", subcore_axis_name="s", num_cores=_NC)

    def _chunks(n):
        off, out = 0, []
        while n:
            c = min(n, CH)
            out.append((off, c))
            off += c
            n -= c
        return out

    @functools.partial(
        pl.kernel,
        out_type=(
            jax.ShapeDtypeStruct((F, B), jnp.float32),
            jax.ShapeDtypeStruct((F, B), jnp.float32),
            jax.ShapeDtypeStruct((Fd, B), jnp.float32),
            jax.ShapeDtypeStruct((B,), jnp.float32),
        ),
        mesh=mesh,
        scratch_types=[
            pltpu.VMEM((B,), jnp.int32),
            pltpu.VMEM((B,), jnp.int32),
            pltpu.VMEM((H1,), jnp.float32),
            pltpu.VMEM((B2,), jnp.float32),
            pltpu.VMEM((B,), jnp.float32),
            pltpu.SemaphoreType.DMA,
            pltpu.SemaphoreType.DMA,
        ],
        compiler_params=pltpu.CompilerParams(needs_layout_passes=False),
    )
    def gather_kernel(user_hbm, item_hbm, bi_hbm, gut_hbm, git_hbm,
                      tut_hbm, gu_tail_hbm, gi_tail_hbm, tu_tail_hbm,
                      gut_out, git_out, tut_out, beta_out,
                      uidx_v, iidx_v, buf_a, buf_b, acc_v, sem_a, sem_b):
        wid = lax.axis_index("s") * _NC + lax.axis_index("c")
        pltpu.sync_copy(user_hbm, uidx_v)
        pltpu.sync_copy(item_hbm, iidx_v)
        lanes = lax.iota(jnp.int32, 16)

        def fire_h1(tab_hbm, k):
            return [pltpu.async_copy(tab_hbm.at[k, pl.ds(o, c)],
                                     buf_a.at[pl.ds(o, c)], sem_a)
                    for o, c in _chunks(H1)]

        def fire_h2(tab_hbm, tail_hbm, k):
            fired = [pltpu.async_copy(tab_hbm.at[k, pl.ds(H1 + o, c)],
                                      buf_b.at[pl.ds(o, c)], sem_b)
                     for o, c in _chunks(H2)]
            fired.append(pltpu.async_copy(
                tail_hbm.at[pl.ds(k * tail_b, tail_b)],
                buf_b.at[pl.ds(H2, tail_b)], sem_b))
            return fired

        def gather_half(idx_ref, buf, lo, size):
            def step(v, _):
                ivec = idx_ref[pl.ds(v * 16, 16)]
                rel = ivec - lo
                sel = (rel >= 0) & (rel < size)
                loc = jnp.where(sel, rel, 0)
                vals = plsc.load_gather(buf, [loc])
                plsc.store_scatter(acc_v, [lanes + v * 16], vals,
                                   mask=sel)
                return 0

            lax.fori_loop(0, B // 16, step, 0)

        rows = ([(gut_hbm, gu_tail_hbm, wid * 2 + jj, uidx_v, gut_out)
                 for jj in range(2)]
                + [(git_hbm, gi_tail_hbm, wid * 2 + jj, iidx_v, git_out)
                   for jj in range(2)]
                + [(tut_hbm, tu_tail_hbm, wid, uidx_v, tut_out)])

        halves = []
        for tab, tail, k, idx_ref, out in rows:
            halves.append(("a", tab, tail, k, idx_ref, None))
            halves.append(("b", tab, tail, k, idx_ref, out))

        def fire(i):
            which, tab, tail, k, _, _ = halves[i]
            if which == "a":
                return fire_h1(tab, k)
            return fire_h2(tab, tail, k)

        pend = {0: fire(0), 1: fire(1)}
        for i in range(len(halves)):
            for c in pend.pop(i):
                c.wait()
            which, tab, tail, k, idx_ref, out = halves[i]
            if which == "a":
                gather_half(idx_ref, buf_a, 0, H1)
            else:
                gather_half(idx_ref, buf_b, H1, B2)
            if i + 2 < len(halves):
                pend[i + 2] = fire(i + 2)
            if out is not None:
                pltpu.sync_copy(acc_v, out.at[k, pl.ds(0, B)])

        nbw = 8
        bslice = B // nbw

        @pl.when(wid >= _NW - nbw)
        def _():
            fired = [pltpu.async_copy(bi_hbm.at[pl.ds(o, c)],
                                      buf_a.at[pl.ds(o, c)], sem_a)
                     for o, c in _chunks(H1)]
            for c in fired:
                c.wait()
            fired = [pltpu.async_copy(bi_hbm.at[pl.ds(H1 + o, c)],
                                      buf_b.at[pl.ds(o, c)], sem_b)
                     for o, c in _chunks(V - H1)]
            for c in fired:
                c.wait()
            q = wid - (_NW - nbw)
            base = q * bslice

            def bstep(v, _):
                ivec = iidx_v[pl.ds(base + v * 16, 16)]
                rel = ivec - H1
                sel = rel >= 0
                loc_a = jnp.where(sel, 0, ivec)
                loc_b = jnp.where(sel, rel, 0)
                va = plsc.load_gather(buf_a, [loc_a])
                vb = plsc.load_gather(buf_b, [loc_b])
                acc_v[pl.ds(v * 16, 16)] = jnp.where(sel, vb, va)
                return 0

            lax.fori_loop(0, bslice // 16, bstep, 0)
            pltpu.sync_copy(acc_v.at[pl.ds(0, bslice)],
                            beta_out.at[pl.ds(base, bslice)])

    return gather_kernel


def _make_sc_beta(B):
    b_per_w = B // _NW
    mesh = plsc.VectorSubcoreMesh(
        core_axis_name="c", subcore_axis_name="s", num_cores=_NC)

    @functools.partial(
        pl.kernel,
        out_type=jax.ShapeDtypeStruct((B, 8), jnp.float32),
        mesh=mesh,
        scratch_types=[
            pltpu.VMEM((b_per_w,), jnp.int32),
            pltpu.VMEM((b_per_w, 8), jnp.float32),
            pltpu.SemaphoreType.DMA,
        ],
        compiler_params=pltpu.CompilerParams(needs_layout_passes=False),
    )
    def beta_kernel(item_hbm, bi_hbm, bi8_out, iidx_v, bi8_v, sem):
        wid = lax.axis_index("s") * _NC + lax.axis_index("c")
        base = wid * b_per_w
        pltpu.sync_copy(item_hbm.at[pl.ds(base, b_per_w)], iidx_v)
        for g in range(b_per_w // 16):
            tvec = iidx_v[pl.ds(g * 16, 16)]
            fired = []
            for j in range(16):
                t = tvec[j]
                t8 = (t // 8) * 8
                i = g * 16 + j
                fired.append(pltpu.async_copy(
                    bi_hbm.at[pl.ds(t8, 8)], bi8_v.at[i], sem))
            for c in fired:
                c.wait()
        pltpu.sync_copy(bi8_v, bi8_out.at[pl.ds(base, b_per_w)])

    return beta_kernel


def _make_tc_matmul(B, K, N, blk):
    def body(feat_ref, ew_ref, r_ref, fcopy_ref):
        r_ref[...] = jnp.dot(feat_ref[...], ew_ref[...],
                             preferred_element_type=jnp.float32)
        fcopy_ref[...] = feat_ref[...]

    return pl.pallas_call(
        body,
        grid=(B // blk,),
        in_specs=[
            pl.BlockSpec((blk, K), lambda b: (b, 0)),
            pl.BlockSpec((K, N), lambda b: (0, 0)),
        ],
        out_specs=[
            pl.BlockSpec((blk, N), lambda b: (b, 0)),
            pl.BlockSpec((blk, K), lambda b: (b, 0)),
        ],
        out_shape=[
            jax.ShapeDtypeStruct((B, N), jnp.float32),
            jax.ShapeDtypeStruct((B, K), jnp.float32),
        ],
        compiler_params=pltpu.CompilerParams(
            dimension_semantics=("arbitrary",)),
    )


def _make_tc_combine(B, F, N, blk):
    def body(r_ref, gut_ref, git_ref, thp_ref, beta_ref, xui_ref):
        xui_ref[...] = (beta_ref[...]
                        + jnp.sum(gut_ref[...] * git_ref[...], axis=0)
                        + jnp.sum(thp_ref[...] * r_ref[...], axis=1))

    return pl.pallas_call(
        body,
        grid=(B // blk,),
        in_specs=[
            pl.BlockSpec((blk, N), lambda b: (b, 0)),
            pl.BlockSpec((F, blk), lambda b: (0, b)),
            pl.BlockSpec((F, blk), lambda b: (0, b)),
            pl.BlockSpec((blk, N), lambda b: (b, 0)),
            pl.BlockSpec((blk,), lambda b: (b,)),
        ],
        out_specs=pl.BlockSpec((blk,), lambda b: (b,)),
        out_shape=jax.ShapeDtypeStruct((B,), jnp.float32),
        compiler_params=pltpu.CompilerParams(
            dimension_semantics=("arbitrary",)),
    )


def kernel(user, item, feature_i, Bi, Gu, Gi, Bp, Tu, E):
    B = user.shape[0]
    K, Fd = E.shape
    V, F = Gu.shape
    N = 64

    V128 = (V // 128) * 128
    gut, git, tut, beta_i = _make_sc_gather(B, F, Fd, V)(
        user, item, Bi, Gu.T, Gi.T, Tu.T,
        Gu.T[:, V128:].reshape(-1), Gi.T[:, V128:].reshape(-1),
        Tu.T[:, V128:].reshape(-1))

    Ew = jnp.concatenate(
        [E, Bp, jnp.zeros((K, N - Fd - 1), jnp.float32)], axis=1)
    r, feat_out = _make_tc_matmul(B, K, N, 512)(feature_i, Ew)

    theta_u = tut.T
    thp = jnp.concatenate(
        [theta_u, jnp.ones((B, 1), jnp.float32),
         jnp.zeros((B, N - Fd - 1), jnp.float32)], axis=1)

    xui = _make_tc_combine(B, F, N, 1024)(r, gut, git, thp, beta_i)

    return (xui, gut.T, git.T, feat_out, theta_u, beta_i)


# speedup vs baseline: 1.0918x; 1.0918x over previous
import functools

import jax
import jax.numpy as jnp
from jax import lax
from jax.experimental import pallas as pl
from jax.experimental.pallas import tpu as pltpu
from jax.experimental.pallas import tpu_sc as plsc

_NC, _NS = 2, 16
_NW = _NC * _NS


def _make_sc_gather(B, F, Fd, V):
    CH = 32768
    V128 = (V // 128) * 128
    n_full = V128 // CH
    tail_a = V128 - n_full * CH
    tail_b = V - V128
    mesh = plsc.VectorSubcoreMesh(
        core_axis_name="c", subcore_axis_name="s", num_cores=_NC)

    @functools.partial(
        pl.kernel,
        out_type=(
            jax.ShapeDtypeStruct((F, B), jnp.float32),
            jax.ShapeDtypeStruct((F, B), jnp.float32),
            jax.ShapeDtypeStruct((Fd, B), jnp.float32),
            jax.ShapeDtypeStruct((B,), jnp.float32),
        ),
        mesh=mesh,
        scratch_types=[
            pltpu.VMEM((B,), jnp.int32),
            pltpu.VMEM((B,), jnp.int32),
            pltpu.VMEM((V,), jnp.float32),
            pltpu.VMEM((B,), jnp.float32),
            pltpu.SemaphoreType.DMA,
        ],
        compiler_params=pltpu.CompilerParams(needs_layout_passes=False),
    )
    def gather_kernel(user_hbm, item_hbm, bi_hbm, gut_hbm, git_hbm,
                      tut_hbm, gu_tail_hbm, gi_tail_hbm, tu_tail_hbm,
                      gut_out, git_out, tut_out, beta_out,
                      uidx_v, iidx_v, slab_v, acc_v, sem):
        wid = lax.axis_index("s") * _NC + lax.axis_index("c")
        pltpu.sync_copy(user_hbm, uidx_v)
        pltpu.sync_copy(item_hbm, iidx_v)

        def stream_row(tab_hbm, tail_hbm, k):
            fired = []
            for c in range(n_full):
                fired.append(pltpu.async_copy(
                    tab_hbm.at[k, pl.ds(c * CH, CH)],
                    slab_v.at[pl.ds(c * CH, CH)], sem))
            if tail_a:
                fired.append(pltpu.async_copy(
                    tab_hbm.at[k, pl.ds(n_full * CH, tail_a)],
                    slab_v.at[pl.ds(n_full * CH, tail_a)], sem))
            if tail_b:
                fired.append(pltpu.async_copy(
                    tail_hbm.at[pl.ds(k * tail_b, tail_b)],
                    slab_v.at[pl.ds(V128, tail_b)], sem))
            for c in fired:
                c.wait()

        def gather_row(idx_ref, out_hbm, k):
            def step(v, _):
                ivec = idx_ref[pl.ds(v * 16, 16)]
                acc_v[pl.ds(v * 16, 16)] = plsc.load_gather(
                    slab_v, [ivec])
                return 0

            lax.fori_loop(0, B // 16, step, 0)
            pltpu.sync_copy(acc_v, out_hbm.at[k, pl.ds(0, B)])

        for jj in range(2):
            k = wid * 2 + jj
            stream_row(gut_hbm, gu_tail_hbm, k)
            gather_row(uidx_v, gut_out, k)
        for jj in range(2):
            k = wid * 2 + jj
            stream_row(git_hbm, gi_tail_hbm, k)
            gather_row(iidx_v, git_out, k)
        stream_row(tut_hbm, tu_tail_hbm, wid)
        gather_row(uidx_v, tut_out, wid)

        nbw = 8
        bslice = B // nbw

        @pl.when(wid >= _NW - nbw)
        def _():
            fired = []
            for c in range(n_full):
                fired.append(pltpu.async_copy(
                    bi_hbm.at[pl.ds(c * CH, CH)],
                    slab_v.at[pl.ds(c * CH, CH)], sem))
            rest = V - n_full * CH
            fired.append(pltpu.async_copy(
                bi_hbm.at[pl.ds(n_full * CH, rest)],
                slab_v.at[pl.ds(n_full * CH, rest)], sem))
            for c in fired:
                c.wait()
            q = wid - (_NW - nbw)
            base = q * bslice

            def bstep(v, _):
                ivec = iidx_v[pl.ds(base + v * 16, 16)]
                acc_v[pl.ds(v * 16, 16)] = plsc.load_gather(
                    slab_v, [ivec])
                return 0

            lax.fori_loop(0, bslice // 16, bstep, 0)
            pltpu.sync_copy(acc_v.at[pl.ds(0, bslice)],
                            beta_out.at[pl.ds(base, bslice)])

    return gather_kernel


def _make_sc_beta(B):
    b_per_w = B // _NW
    mesh = plsc.VectorSubcoreMesh(
        core_axis_name="c", subcore_axis_name="s", num_cores=_NC)

    @functools.partial(
        pl.kernel,
        out_type=jax.ShapeDtypeStruct((B, 8), jnp.float32),
        mesh=mesh,
        scratch_types=[
            pltpu.VMEM((b_per_w,), jnp.int32),
            pltpu.VMEM((b_per_w, 8), jnp.float32),
            pltpu.SemaphoreType.DMA,
        ],
        compiler_params=pltpu.CompilerParams(needs_layout_passes=False),
    )
    def beta_kernel(item_hbm, bi_hbm, bi8_out, iidx_v, bi8_v, sem):
        wid = lax.axis_index("s") * _NC + lax.axis_index("c")
        base = wid * b_per_w
        pltpu.sync_copy(item_hbm.at[pl.ds(base, b_per_w)], iidx_v)
        for g in range(b_per_w // 16):
            tvec = iidx_v[pl.ds(g * 16, 16)]
            fired = []
            for j in range(16):
                t = tvec[j]
                t8 = (t // 8) * 8
                i = g * 16 + j
                fired.append(pltpu.async_copy(
                    bi_hbm.at[pl.ds(t8, 8)], bi8_v.at[i], sem))
            for c in fired:
                c.wait()
        pltpu.sync_copy(bi8_v, bi8_out.at[pl.ds(base, b_per_w)])

    return beta_kernel


def _make_tc_matmul(B, K, N, blk):
    def body(feat_ref, ew_ref, r_ref, fcopy_ref):
        r_ref[...] = jnp.dot(feat_ref[...], ew_ref[...],
                             preferred_element_type=jnp.float32)
        fcopy_ref[...] = feat_ref[...]

    return pl.pallas_call(
        body,
        grid=(B // blk,),
        in_specs=[
            pl.BlockSpec((blk, K), lambda b: (b, 0)),
            pl.BlockSpec((K, N), lambda b: (0, 0)),
        ],
        out_specs=[
            pl.BlockSpec((blk, N), lambda b: (b, 0)),
            pl.BlockSpec((blk, K), lambda b: (b, 0)),
        ],
        out_shape=[
            jax.ShapeDtypeStruct((B, N), jnp.float32),
            jax.ShapeDtypeStruct((B, K), jnp.float32),
        ],
        compiler_params=pltpu.CompilerParams(
            dimension_semantics=("arbitrary",)),
    )


def _make_tc_combine(B, F, Fd, N, blk):
    def body(r_ref, gut_ref, git_ref, tut_ref, beta_ref, xui_ref):
        tu = jnp.transpose(tut_ref[...])
        r = r_ref[...]
        xui_ref[...] = (beta_ref[...]
                        + jnp.sum(gut_ref[...] * git_ref[...], axis=0)
                        + jnp.sum(tu * r[:, :Fd], axis=1)
                        + r[:, Fd])

    return pl.pallas_call(
        body,
        grid=(B // blk,),
        in_specs=[
            pl.BlockSpec((blk, N), lambda b: (b, 0)),
            pl.BlockSpec((F, blk), lambda b: (0, b)),
            pl.BlockSpec((F, blk), lambda b: (0, b)),
            pl.BlockSpec((Fd, blk), lambda b: (0, b)),
            pl.BlockSpec((blk,), lambda b: (b,)),
        ],
        out_specs=pl.BlockSpec((blk,), lambda b: (b,)),
        out_shape=jax.ShapeDtypeStruct((B,), jnp.float32),
        compiler_params=pltpu.CompilerParams(
            dimension_semantics=("arbitrary",)),
    )


def kernel(user, item, feature_i, Bi, Gu, Gi, Bp, Tu, E):
    B = user.shape[0]
    K, Fd = E.shape
    V, F = Gu.shape
    N = 64

    V128 = (V // 128) * 128
    gut, git, tut, beta_i = _make_sc_gather(B, F, Fd, V)(
        user, item, Bi, Gu.T, Gi.T, Tu.T,
        Gu.T[:, V128:].reshape(-1), Gi.T[:, V128:].reshape(-1),
        Tu.T[:, V128:].reshape(-1))

    Ew = jnp.concatenate(
        [E, Bp, jnp.zeros((K, N - Fd - 1), jnp.float32)], axis=1)
    r, feat_out = _make_tc_matmul(B, K, N, 1024)(feature_i, Ew)

    xui = _make_tc_combine(B, F, Fd, N, 1024)(r, gut, git, tut, beta_i)

    return (xui, gut.T, git.T, feat_out, tut.T, beta_i)
